# baseline (speedup 1.0000x reference)
import functools

import jax
import jax.numpy as jnp
from jax import lax
from jax.experimental import pallas as pl
from jax.experimental.pallas import tpu as pltpu
from jax.experimental.pallas import tpu_sc as plsc

N = 10000
E = 320000
F = 128
H = 128

NC = 2
NS = 16
NW = NC * NS
CHUNK = 512
NCH = E // CHUNK
CPW = NCH // NW
RBLK = 80

_MESH = plsc.VectorSubcoreMesh(core_axis_name="c", subcore_axis_name="s")
_SC_PARAMS = pltpu.CompilerParams(use_tc_tiling_on_sc=False)


@functools.partial(
    pl.kernel,
    out_type=jax.ShapeDtypeStruct((NC, N), jnp.float32),
    mesh=_MESH,
    compiler_params=_SC_PARAMS,
    scratch_types=[
        pltpu.VMEM((CPW + 1, CHUNK), jnp.int32),
        pltpu.VMEM((CHUNK,), jnp.float32),
        pltpu.VMEM((1024,), jnp.float32),
        pltpu.VMEM_SHARED((N,), jnp.float32),
        pltpu.SemaphoreType.DMA,
    ],
)
def _deg_kernel(edges_hbm, deg_out, idx_v, ones_v, zb_v, deg_sh, sem):
    c = lax.axis_index("c")
    s = lax.axis_index("s")
    wid = s * NC + c

    def _z(i, _):
        zb_v[pl.ds(i * 16, 16)] = jnp.zeros((16,), jnp.float32)
        return 0
    lax.fori_loop(0, 64, _z, 0)
    for i in range(CHUNK // 16):
        ones_v[pl.ds(i * 16, 16)] = jnp.ones((16,), jnp.float32)

    @pl.when(s < 10)
    def _():
        pltpu.sync_copy(zb_v.at[pl.ds(0, 1000)], deg_sh.at[pl.ds(s * 1000, 1000)])
    plsc.subcore_barrier()

    pltpu.sync_copy(edges_hbm.at[1, pl.ds(wid * CPW, CPW)],
                    idx_v.at[pl.ds(0, CPW)])

    @pl.when(wid < NCH - NW * CPW)
    def _():
        pltpu.sync_copy(edges_hbm.at[1, pl.ds(NW * CPW + wid, 1)],
                        idx_v.at[pl.ds(CPW, 1)])

    def _chunk(g, _):
        pltpu.sync_copy(ones_v, deg_sh.at[idx_v.at[g]], add=True)
        return 0
    lax.fori_loop(0, CPW, _chunk, 0)

    @pl.when(wid < NCH - NW * CPW)
    def _():
        pltpu.sync_copy(ones_v, deg_sh.at[idx_v.at[CPW]], add=True)
    plsc.subcore_barrier()

    @pl.when(s < 10)
    def _():
        pltpu.sync_copy(deg_sh.at[pl.ds(s * 1000, 1000)],
                        deg_out.at[c, pl.ds(s * 1000, 1000)])


RB = 1000


def _dense_body(x_ref, w_ref, parts_ref, y_ref):
    deg = 1.0 + parts_ref[0] + parts_ref[1]
    dis = lax.rsqrt(deg)
    xw = jnp.dot(x_ref[...], w_ref[...], preferred_element_type=jnp.float32)
    y_ref[...] = (xw * dis).astype(jnp.bfloat16)


_dense = pl.pallas_call(
    _dense_body,
    grid=(N // RB,),
    in_specs=[
        pl.BlockSpec((RB, F), lambda i: (i, 0)),
        pl.BlockSpec((F, H), lambda i: (0, 0)),
        pl.BlockSpec((NC, RB, 1), lambda i: (0, i, 0)),
    ],
    out_specs=pl.BlockSpec((RB, H), lambda i: (i, 0)),
    out_shape=jax.ShapeDtypeStruct((N, H), jnp.bfloat16),
)


@functools.partial(
    pl.kernel,
    out_type=jax.ShapeDtypeStruct((NC, N, H), jnp.bfloat16),
    mesh=_MESH,
    compiler_params=_SC_PARAMS,
    scratch_types=[
        pltpu.VMEM((CPW + 1, CHUNK), jnp.int32),
        pltpu.VMEM((CPW + 1, CHUNK), jnp.int32),
        pltpu.VMEM((2, CHUNK, H), jnp.bfloat16),
        pltpu.VMEM_SHARED((N, H), jnp.bfloat16),
        pltpu.SemaphoreType.DMA,
        pltpu.SemaphoreType.DMA,
    ],
)
def _msg_kernel(y_hbm, edges_hbm, acc_out, sidx, didx, rows, acc_sh,
                sem0, sem1):
    c = lax.axis_index("c")
    s = lax.axis_index("s")
    wid = s * NC + c

    def _z(t, _):
        i = t // 4
        j = t % 4
        rows[0, i, pl.ds(j * 32, 32)] = jnp.zeros((32,), jnp.bfloat16)
        return 0
    lax.fori_loop(0, CHUNK * 4, _z, 0)

    for k in range(8):
        blk = s * 8 + k

        @pl.when(blk < N // RBLK)
        def _():
            pltpu.sync_copy(rows.at[0, pl.ds(0, RBLK)],
                            acc_sh.at[pl.ds(blk * RBLK, RBLK)])
    plsc.subcore_barrier()

    pltpu.sync_copy(edges_hbm.at[0, pl.ds(wid * CPW, CPW)],
                    sidx.at[pl.ds(0, CPW)])
    pltpu.sync_copy(edges_hbm.at[1, pl.ds(wid * CPW, CPW)],
                    didx.at[pl.ds(0, CPW)])

    @pl.when(wid < NCH - NW * CPW)
    def _():
        pltpu.sync_copy(edges_hbm.at[0, pl.ds(NW * CPW + wid, 1)],
                        sidx.at[pl.ds(CPW, 1)])
        pltpu.sync_copy(edges_hbm.at[1, pl.ds(NW * CPW + wid, 1)],
                        didx.at[pl.ds(CPW, 1)])

    pltpu.sync_copy(y_hbm.at[sidx.at[0]], rows.at[0])

    def _pair(i, _):
        g = 2 * i
        d1 = pltpu.async_copy(y_hbm.at[sidx.at[g + 1]], rows.at[1], sem1)
        pltpu.sync_copy(rows.at[0], acc_sh.at[didx.at[g]], add=True)
        d1.wait()
        d2 = pltpu.async_copy(y_hbm.at[sidx.at[g + 2]], rows.at[0], sem0)
        pltpu.sync_copy(rows.at[1], acc_sh.at[didx.at[g + 1]], add=True)
        d2.wait()
        return 0
    _P = (CPW - 2) // 2
    lax.fori_loop(0, _P, _pair, 0)

    _t = 2 * _P
    d1 = pltpu.async_copy(y_hbm.at[sidx.at[_t + 1]], rows.at[1], sem1)
    pltpu.sync_copy(rows.at[0], acc_sh.at[didx.at[_t]], add=True)
    d1.wait()
    if CPW - _t == 2:
        pltpu.sync_copy(rows.at[1], acc_sh.at[didx.at[_t + 1]], add=True)
    else:
        d2 = pltpu.async_copy(y_hbm.at[sidx.at[_t + 2]], rows.at[0], sem0)
        pltpu.sync_copy(rows.at[1], acc_sh.at[didx.at[_t + 1]], add=True)
        d2.wait()
        pltpu.sync_copy(rows.at[0], acc_sh.at[didx.at[_t + 2]], add=True)

    @pl.when(wid < NCH - NW * CPW)
    def _():
        pltpu.sync_copy(y_hbm.at[sidx.at[CPW]], rows.at[0])
        pltpu.sync_copy(rows.at[0], acc_sh.at[didx.at[CPW]], add=True)
    plsc.subcore_barrier()

    for k in range(8):
        blk = s * 8 + k

        @pl.when(blk < N // RBLK)
        def _():
            pltpu.sync_copy(acc_sh.at[pl.ds(blk * RBLK, RBLK)],
                            acc_out.at[c, pl.ds(blk * RBLK, RBLK)])


def _combine_body(acc_ref, y_ref, parts_ref, b_ref, out_ref):
    deg = 1.0 + parts_ref[0] + parts_ref[1]
    dis = lax.rsqrt(deg)
    msgs = (acc_ref[0].astype(jnp.float32) + acc_ref[1].astype(jnp.float32)
            + y_ref[...].astype(jnp.float32))
    out_ref[...] = dis * msgs + b_ref[...][None, :]


_combine = pl.pallas_call(
    _combine_body,
    grid=(N // RB,),
    in_specs=[
        pl.BlockSpec((NC, RB, H), lambda i: (0, i, 0)),
        pl.BlockSpec((RB, H), lambda i: (i, 0)),
        pl.BlockSpec((NC, RB, 1), lambda i: (0, i, 0)),
        pl.BlockSpec((H,), lambda i: (0,)),
    ],
    out_specs=pl.BlockSpec((RB, H), lambda i: (i, 0)),
    out_shape=jax.ShapeDtypeStruct((N, H), jnp.float32),
)


def kernel(x, edge_index, W, b):
    edges3 = edge_index.astype(jnp.int32).reshape(2, NCH, CHUNK)

    parts = _deg_kernel(edges3)
    parts3 = parts.reshape(NC, N, 1)
    y = _dense(x, W, parts3)
    accs = _msg_kernel(y, edges3)
    return _combine(accs, y, parts3, b)


# speedup vs baseline: 1.0571x; 1.0571x over previous
import functools

import jax
import jax.numpy as jnp
from jax import lax
from jax.experimental import pallas as pl
from jax.experimental.pallas import tpu as pltpu
from jax.experimental.pallas import tpu_sc as plsc

N = 10000
E = 320000
F = 128
H = 128

NC = 2
NS = 16
NW = NC * NS
CHUNK = 512
NCH = E // CHUNK
CPW = NCH // NW
RBLK = 80

_MESH = plsc.VectorSubcoreMesh(core_axis_name="c", subcore_axis_name="s")
_SC_PARAMS = pltpu.CompilerParams(use_tc_tiling_on_sc=False)


@functools.partial(
    pl.kernel,
    out_type=jax.ShapeDtypeStruct((NC, N), jnp.float32),
    mesh=_MESH,
    compiler_params=_SC_PARAMS,
    scratch_types=[
        pltpu.VMEM((CPW + 1, CHUNK), jnp.int32),
        pltpu.VMEM((CHUNK,), jnp.float32),
        pltpu.VMEM((1024,), jnp.float32),
        pltpu.VMEM_SHARED((N,), jnp.float32),
        pltpu.SemaphoreType.DMA,
    ],
)
def _deg_kernel(edges_hbm, deg_out, idx_v, ones_v, zb_v, deg_sh, sem):
    c = lax.axis_index("c")
    s = lax.axis_index("s")
    wid = s * NC + c

    def _z(i, _):
        zb_v[pl.ds(i * 16, 16)] = jnp.zeros((16,), jnp.float32)
        return 0
    lax.fori_loop(0, 64, _z, 0)
    for i in range(CHUNK // 16):
        ones_v[pl.ds(i * 16, 16)] = jnp.ones((16,), jnp.float32)

    @pl.when(s < 10)
    def _():
        pltpu.sync_copy(zb_v.at[pl.ds(0, 1000)], deg_sh.at[pl.ds(s * 1000, 1000)])
    plsc.subcore_barrier()

    pltpu.sync_copy(edges_hbm.at[1, pl.ds(wid * CPW, CPW)],
                    idx_v.at[pl.ds(0, CPW)])

    @pl.when(wid < NCH - NW * CPW)
    def _():
        pltpu.sync_copy(edges_hbm.at[1, pl.ds(NW * CPW + wid, 1)],
                        idx_v.at[pl.ds(CPW, 1)])

    def _chunk(g, _):
        pltpu.sync_copy(ones_v, deg_sh.at[idx_v.at[g]], add=True)
        return 0
    lax.fori_loop(0, CPW, _chunk, 0)

    @pl.when(wid < NCH - NW * CPW)
    def _():
        pltpu.sync_copy(ones_v, deg_sh.at[idx_v.at[CPW]], add=True)
    plsc.subcore_barrier()

    @pl.when(s < 10)
    def _():
        pltpu.sync_copy(deg_sh.at[pl.ds(s * 1000, 1000)],
                        deg_out.at[c, pl.ds(s * 1000, 1000)])


RB = 1000


def _dense_body(x_ref, w_ref, parts_ref, y_ref):
    deg = 1.0 + parts_ref[0] + parts_ref[1]
    dis = lax.rsqrt(deg)
    xw = jnp.dot(x_ref[...], w_ref[...], preferred_element_type=jnp.float32)
    y_ref[...] = (xw * dis).astype(jnp.bfloat16)


_dense = pl.pallas_call(
    _dense_body,
    grid=(N // RB,),
    in_specs=[
        pl.BlockSpec((RB, F), lambda i: (i, 0)),
        pl.BlockSpec((F, H), lambda i: (0, 0)),
        pl.BlockSpec((NC, RB, 1), lambda i: (0, i, 0)),
    ],
    out_specs=pl.BlockSpec((RB, H), lambda i: (i, 0)),
    out_shape=jax.ShapeDtypeStruct((N, H), jnp.bfloat16),
)


@functools.partial(
    pl.kernel,
    out_type=jax.ShapeDtypeStruct((N, NC * H), jnp.bfloat16),
    mesh=_MESH,
    compiler_params=_SC_PARAMS,
    scratch_types=[
        pltpu.VMEM((CPW + 1, CHUNK), jnp.int32),
        pltpu.VMEM((CPW + 1, CHUNK), jnp.int32),
        pltpu.VMEM((2, CHUNK, H), jnp.bfloat16),
        pltpu.VMEM_SHARED((N, H), jnp.bfloat16),
        pltpu.SemaphoreType.DMA,
        pltpu.SemaphoreType.DMA,
    ],
)
def _msg_kernel(y_hbm, edges_hbm, acc_out, sidx, didx, rows, acc_sh,
                sem0, sem1):
    c = lax.axis_index("c")
    s = lax.axis_index("s")
    wid = s * NC + c

    def _z(t, _):
        i = t // 4
        j = t % 4
        rows[0, i, pl.ds(j * 32, 32)] = jnp.zeros((32,), jnp.bfloat16)
        return 0
    lax.fori_loop(0, CHUNK * 4, _z, 0)

    for k in range(8):
        blk = s * 8 + k

        @pl.when(blk < N // RBLK)
        def _():
            pltpu.sync_copy(rows.at[0, pl.ds(0, RBLK)],
                            acc_sh.at[pl.ds(blk * RBLK, RBLK)])
    plsc.subcore_barrier()

    pltpu.sync_copy(edges_hbm.at[0, pl.ds(wid * CPW, CPW)],
                    sidx.at[pl.ds(0, CPW)])
    pltpu.sync_copy(edges_hbm.at[1, pl.ds(wid * CPW, CPW)],
                    didx.at[pl.ds(0, CPW)])

    @pl.when(wid < NCH - NW * CPW)
    def _():
        pltpu.sync_copy(edges_hbm.at[0, pl.ds(NW * CPW + wid, 1)],
                        sidx.at[pl.ds(CPW, 1)])
        pltpu.sync_copy(edges_hbm.at[1, pl.ds(NW * CPW + wid, 1)],
                        didx.at[pl.ds(CPW, 1)])

    pltpu.sync_copy(y_hbm.at[sidx.at[0]], rows.at[0])

    def _pair(i, _):
        g = 2 * i
        d1 = pltpu.async_copy(y_hbm.at[sidx.at[g + 1]], rows.at[1], sem1)
        pltpu.sync_copy(rows.at[0], acc_sh.at[didx.at[g]], add=True)
        d1.wait()
        d2 = pltpu.async_copy(y_hbm.at[sidx.at[g + 2]], rows.at[0], sem0)
        pltpu.sync_copy(rows.at[1], acc_sh.at[didx.at[g + 1]], add=True)
        d2.wait()
        return 0
    _P = (CPW - 2) // 2
    lax.fori_loop(0, _P, _pair, 0)

    _t = 2 * _P
    d1 = pltpu.async_copy(y_hbm.at[sidx.at[_t + 1]], rows.at[1], sem1)
    pltpu.sync_copy(rows.at[0], acc_sh.at[didx.at[_t]], add=True)
    d1.wait()
    if CPW - _t == 2:
        pltpu.sync_copy(rows.at[1], acc_sh.at[didx.at[_t + 1]], add=True)
    else:
        d2 = pltpu.async_copy(y_hbm.at[sidx.at[_t + 2]], rows.at[0], sem0)
        pltpu.sync_copy(rows.at[1], acc_sh.at[didx.at[_t + 1]], add=True)
        d2.wait()
        pltpu.sync_copy(rows.at[0], acc_sh.at[didx.at[_t + 2]], add=True)

    @pl.when(wid < NCH - NW * CPW)
    def _():
        pltpu.sync_copy(y_hbm.at[sidx.at[CPW]], rows.at[0])
        pltpu.sync_copy(rows.at[0], acc_sh.at[didx.at[CPW]], add=True)
    plsc.subcore_barrier()

    for k in range(8):
        blk = s * 8 + k

        @pl.when(blk < N // RBLK)
        def _():
            pltpu.sync_copy(acc_sh.at[pl.ds(blk * RBLK, RBLK)],
                            acc_out.at[pl.ds(blk * RBLK, RBLK),
                                       pl.ds(c * H, H)])


def _combine_body(acc_ref, y_ref, parts_ref, b_ref, out_ref):
    deg = 1.0 + parts_ref[0] + parts_ref[1]
    dis = lax.rsqrt(deg)
    msgs = (acc_ref[:, :H].astype(jnp.float32)
            + acc_ref[:, H:].astype(jnp.float32)
            + y_ref[...].astype(jnp.float32))
    out_ref[...] = dis * msgs + b_ref[...][None, :]


_combine = pl.pallas_call(
    _combine_body,
    grid=(N // RB,),
    in_specs=[
        pl.BlockSpec((RB, NC * H), lambda i: (i, 0)),
        pl.BlockSpec((RB, H), lambda i: (i, 0)),
        pl.BlockSpec((NC, RB, 1), lambda i: (0, i, 0)),
        pl.BlockSpec((H,), lambda i: (0,)),
    ],
    out_specs=pl.BlockSpec((RB, H), lambda i: (i, 0)),
    out_shape=jax.ShapeDtypeStruct((N, H), jnp.float32),
)


def kernel(x, edge_index, W, b):
    edges3 = edge_index.astype(jnp.int32).reshape(2, NCH, CHUNK)

    parts = _deg_kernel(edges3)
    parts3 = parts.reshape(NC, N, 1)
    y = _dense(x, W, parts3)
    accs = _msg_kernel(y, edges3)
    return _combine(accs, y, parts3, b)
